# baseline (speedup 1.0000x reference)
import jax
import jax.numpy as jnp
from jax.experimental import pallas as pl
from jax.experimental.pallas import tpu as pltpu

_T = 8192
_SHIFT = 1000
_ROWS = 16 * 128
_BLOCK_ROWS = 64
_GRID = _ROWS // _BLOCK_ROWS


def _roll_body(x_ref, o_ref):
    o_ref[...] = pltpu.roll(x_ref[...], _SHIFT, axis=1)


@jax.jit
def kernel(x):
    rows = x.reshape(_ROWS, _T)
    out = pl.pallas_call(
        _roll_body,
        grid=(_GRID,),
        in_specs=[pl.BlockSpec((_BLOCK_ROWS, _T), lambda i: (i, 0))],
        out_specs=pl.BlockSpec((_BLOCK_ROWS, _T), lambda i: (i, 0)),
        out_shape=jax.ShapeDtypeStruct((_ROWS, _T), jnp.float32),
    )(rows)
    return out.reshape(x.shape)


# speedup vs baseline: 1.1740x; 1.1740x over previous
import jax
import jax.numpy as jnp
from jax.experimental import pallas as pl
from jax.experimental.pallas import tpu as pltpu

_T = 8192
_SHIFT = 1000
_ROWS = 16 * 128
_BLOCK_ROWS = 256
_GRID = _ROWS // _BLOCK_ROWS


def _roll_body(x_ref, o_ref):
    o_ref[...] = pltpu.roll(x_ref[...], _SHIFT, axis=1)


@jax.jit
def kernel(x):
    rows = x.reshape(_ROWS, _T)
    out = pl.pallas_call(
        _roll_body,
        grid=(_GRID,),
        in_specs=[pl.BlockSpec((_BLOCK_ROWS, _T), lambda i: (i, 0))],
        out_specs=pl.BlockSpec((_BLOCK_ROWS, _T), lambda i: (i, 0)),
        out_shape=jax.ShapeDtypeStruct((_ROWS, _T), jnp.float32),
    )(rows)
    return out.reshape(x.shape)
